# baseline (speedup 1.0000x reference)
import jax
import jax.numpy as jnp
from jax.experimental import pallas as pl
from jax.experimental.pallas import tpu as pltpu

N = 2048
M = 2048
TM = 512
NT = M // TM


def _chamfer_body(xs_ref, yt_ref, o_ref):
    b = pl.program_id(0)
    nb = pl.num_programs(0)

    xf = xs_ref[...].astype(jnp.float32) * -0.5
    x2 = jnp.sum(xf * xf, axis=1, keepdims=True)
    yf = yt_ref[...].astype(jnp.float32)
    y2 = jnp.sum(yf * yf, axis=0, keepdims=True)

    m1 = None
    s2acc = None
    for t in range(NT):
        w = jax.lax.dot_general(
            xs_ref[...],
            yt_ref[:, pl.ds(t * TM, TM)],
            (((1,), (0,)), ((), ())),
            preferred_element_type=jnp.float32,
        )
        ttile = w + y2[0:1, t * TM : (t + 1) * TM]
        m1t = jnp.min(ttile, axis=1, keepdims=True)
        m1 = m1t if m1 is None else jnp.minimum(m1, m1t)
        ct = jnp.min(ttile + x2, axis=0, keepdims=True)
        s2acc = ct if s2acc is None else s2acc + ct
    s1 = jnp.sum(m1 + x2)
    s2 = jnp.sum(s2acc)
    step = (s1 + s2) * (1.0 / (N * nb))

    @pl.when(b == 0)
    def _init():
        o_ref[...] = jnp.zeros_like(o_ref)

    o_ref[...] += step


@jax.jit
def kernel(receptive_pc, decoder_pc):
    b = receptive_pc.shape[0]
    xs = (-2.0 * receptive_pc).astype(jnp.bfloat16)
    yt = jnp.swapaxes(decoder_pc, 1, 2).astype(jnp.bfloat16)
    out = pl.pallas_call(
        _chamfer_body,
        grid=(b,),
        in_specs=[
            pl.BlockSpec((None, N, 3), lambda i: (i, 0, 0)),
            pl.BlockSpec((None, 3, M), lambda i: (i, 0, 0)),
        ],
        out_specs=pl.BlockSpec((1, 1), lambda i: (0, 0)),
        out_shape=jax.ShapeDtypeStruct((1, 1), jnp.float32),
        compiler_params=pltpu.CompilerParams(
            dimension_semantics=("arbitrary",),
        ),
    )(xs, yt)
    return out.reshape(())
